# initial kernel scaffold (unmeasured)
import jax
import jax.numpy as jnp
from jax import lax
from jax.experimental import pallas as pl
from jax.experimental.pallas import tpu as pltpu


def kernel(
    x,
):
    def body(*refs):
        pass

    out_shape = jax.ShapeDtypeStruct(..., jnp.float32)
    return pl.pallas_call(body, out_shape=out_shape)(...)


# baseline (device time: 1165439 ns/iter reference)
import jax
from jax import lax
from jax.experimental import pallas as pl
from jax.experimental.pallas import tpu as pltpu

P = 4
NDIR = 2


def kernel(x):
    M, N = x.shape
    CH = M // (NDIR * P)
    HALF = P * CH

    def body(x_hbm, out_hbm, comm, stage, send_sems, recv_sems, local_sem):
        my_x = lax.axis_index("x")
        my_y = lax.axis_index("y")
        my_z = lax.axis_index("z")

        def chunk(ref, c, d):
            return ref.at[pl.ds(d * HALF + c * CH, CH), :]

        def to_y(y):
            return (my_x, y, my_z)

        right = (my_y + 1) % P
        left = (my_y + P - 1) % P
        barrier = pltpu.get_barrier_semaphore()
        for nbr in (right, left):
            pl.semaphore_signal(
                barrier, inc=1, device_id=to_y(nbr),
                device_id_type=pl.DeviceIdType.MESH,
            )
        pl.semaphore_wait(barrier, 2)

        sends = []

        for s in range(P - 1):
            started = []
            for d in range(NDIR):
                sgn = 1 if d == 0 else -1
                c_send = (my_y - sgn * s + 2 * P) % P
                src = chunk(x_hbm, c_send, d) if s == 0 else comm.at[d, s - 1]
                rdma = pltpu.make_async_remote_copy(
                    src_ref=src,
                    dst_ref=comm.at[d, s],
                    send_sem=send_sems.at[d, s],
                    recv_sem=recv_sems.at[d, s],
                    device_id=to_y((my_y + sgn + P) % P),
                    device_id_type=pl.DeviceIdType.MESH,
                )
                rdma.start()
                started.append(rdma)
                sends.append(rdma)
            for d in range(NDIR):
                sgn = 1 if d == 0 else -1
                started[d].wait_recv()
                c_recv = (my_y - sgn * (s + 1) + 2 * P) % P
                cp = pltpu.make_async_copy(
                    chunk(x_hbm, c_recv, d), stage, local_sem
                )
                cp.start()
                cp.wait()
                comm[d, s] = comm[d, s] + stage[:, :]

        for d in range(NDIR):
            sgn = 1 if d == 0 else -1
            c_own = (my_y + sgn + P) % P
            cp = pltpu.make_async_copy(
                comm.at[d, P - 2], chunk(out_hbm, c_own, d), local_sem
            )
            cp.start()
            cp.wait()

        for t in range(P - 1):
            started = []
            for d in range(NDIR):
                sgn = 1 if d == 0 else -1
                c_send = (my_y + sgn - sgn * t + 2 * P) % P
                src = comm.at[d, P - 2] if t == 0 else chunk(out_hbm, c_send, d)
                rdma = pltpu.make_async_remote_copy(
                    src_ref=src,
                    dst_ref=chunk(out_hbm, c_send, d),
                    send_sem=send_sems.at[d, P - 1 + t],
                    recv_sem=recv_sems.at[d, P - 1 + t],
                    device_id=to_y((my_y + sgn + P) % P),
                    device_id_type=pl.DeviceIdType.MESH,
                )
                rdma.start()
                started.append(rdma)
                sends.append(rdma)
            for d in range(NDIR):
                started[d].wait_recv()

        for rdma in sends:
            rdma.wait_send()

    return pl.pallas_call(
        body,
        out_shape=jax.ShapeDtypeStruct((M, N), x.dtype),
        in_specs=[pl.BlockSpec(memory_space=pltpu.MemorySpace.HBM)],
        out_specs=pl.BlockSpec(memory_space=pltpu.MemorySpace.HBM),
        scratch_shapes=[
            pltpu.VMEM((NDIR, P - 1, CH, N), x.dtype),
            pltpu.VMEM((CH, N), x.dtype),
            pltpu.SemaphoreType.DMA((NDIR, 2 * (P - 1))),
            pltpu.SemaphoreType.DMA((NDIR, 2 * (P - 1))),
            pltpu.SemaphoreType.DMA,
        ],
        compiler_params=pltpu.CompilerParams(
            collective_id=0, vmem_limit_bytes=60 * 1024 * 1024
        ),
    )(x)


# device time: 745393 ns/iter; 1.5635x vs baseline; 1.5635x over previous
import jax
import jax.numpy as jnp
from jax import lax
from jax.experimental import pallas as pl
from jax.experimental.pallas import tpu as pltpu

R = 8
NDIR = 2


def kernel(x):
    M, N = x.shape
    CH = M // (NDIR * R)
    HALF = R * CH

    def body(x_hbm, out_hbm, comm, stage, send_sems, recv_sems, local_sem):
        my_x = lax.axis_index("x")
        my_y = lax.axis_index("y")
        my_z = lax.axis_index("z")
        pos = jnp.where(my_x == 0, my_y, 7 - my_y)

        def coords_of(p):
            p = (p + 2 * R) % R
            px = p // 4
            py = jnp.where(px == 0, p, 7 - p)
            return (px, py, my_z)

        def chunk(ref, c, d):
            c = (c + 2 * R) % R
            return ref.at[pl.ds(d * HALF + c * CH, CH), :]

        barrier = pltpu.get_barrier_semaphore()
        for sgn in (1, -1):
            pl.semaphore_signal(
                barrier, inc=1, device_id=coords_of(pos + sgn),
                device_id_type=pl.DeviceIdType.MESH,
            )
        pl.semaphore_wait(barrier, 2)

        sends = []

        for s in range(R - 1):
            started = []
            for d in range(NDIR):
                sgn = 1 if d == 0 else -1
                src = (
                    chunk(x_hbm, pos, d) if s == 0 else comm.at[d, s - 1]
                )
                rdma = pltpu.make_async_remote_copy(
                    src_ref=src,
                    dst_ref=comm.at[d, s],
                    send_sem=send_sems.at[d, s],
                    recv_sem=recv_sems.at[d, s],
                    device_id=coords_of(pos + sgn),
                    device_id_type=pl.DeviceIdType.MESH,
                )
                rdma.start()
                started.append(rdma)
                sends.append(rdma)
            for d in range(NDIR):
                sgn = 1 if d == 0 else -1
                started[d].wait_recv()
                cp = pltpu.make_async_copy(
                    chunk(x_hbm, pos - sgn * (s + 1), d), stage, local_sem
                )
                cp.start()
                cp.wait()
                if s == 0:
                    comm[d, s] = (comm[d, s] + stage[:, :]) * 0.5
                else:
                    comm[d, s] = comm[d, s] + stage[:, :] * 0.5

        for d in range(NDIR):
            sgn = 1 if d == 0 else -1
            cp = pltpu.make_async_copy(
                comm.at[d, R - 2], chunk(out_hbm, pos + sgn, d), local_sem
            )
            cp.start()
            cp.wait()

        for t in range(R - 1):
            started = []
            for d in range(NDIR):
                sgn = 1 if d == 0 else -1
                c_send = pos + sgn - sgn * t
                src = comm.at[d, R - 2] if t == 0 else chunk(out_hbm, c_send, d)
                rdma = pltpu.make_async_remote_copy(
                    src_ref=src,
                    dst_ref=chunk(out_hbm, c_send, d),
                    send_sem=send_sems.at[d, R - 1 + t],
                    recv_sem=recv_sems.at[d, R - 1 + t],
                    device_id=coords_of(pos + sgn),
                    device_id_type=pl.DeviceIdType.MESH,
                )
                rdma.start()
                started.append(rdma)
                sends.append(rdma)
            for d in range(NDIR):
                started[d].wait_recv()

        for rdma in sends:
            rdma.wait_send()

    n_steps = 2 * (R - 1)
    return pl.pallas_call(
        body,
        out_shape=jax.ShapeDtypeStruct((M, N), x.dtype),
        in_specs=[pl.BlockSpec(memory_space=pltpu.MemorySpace.HBM)],
        out_specs=pl.BlockSpec(memory_space=pltpu.MemorySpace.HBM),
        scratch_shapes=[
            pltpu.VMEM((NDIR, R - 1, CH, N), x.dtype),
            pltpu.VMEM((CH, N), x.dtype),
            pltpu.SemaphoreType.DMA((NDIR, n_steps)),
            pltpu.SemaphoreType.DMA((NDIR, n_steps)),
            pltpu.SemaphoreType.DMA,
        ],
        compiler_params=pltpu.CompilerParams(
            collective_id=0, vmem_limit_bytes=63 * 1024 * 1024
        ),
    )(x)


# device time: 585911 ns/iter; 1.9891x vs baseline; 1.2722x over previous
import jax
import jax.numpy as jnp
from jax import lax
from jax.experimental import pallas as pl
from jax.experimental.pallas import tpu as pltpu

R = 8
NDIR = 2


def kernel(x):
    M, N = x.shape
    CH = M // (2 * NDIR * R)
    HALF = R * CH
    PLANE = NDIR * HALF

    def body(x_hbm, out_hbm, comm, stage, send_sems, recv_sems,
             zsend_sems, zrecv_sems, local_sems):
        my_x = lax.axis_index("x")
        my_y = lax.axis_index("y")
        my_z = lax.axis_index("z")
        pos = jnp.where(my_x == 0, my_y, 7 - my_y)
        zp = my_z % 2
        base = zp * PLANE
        partner = (my_x, my_y, my_z + 1 - 2 * zp)

        def coords_of(p):
            p = (p + 2 * R) % R
            px = p // 4
            py = jnp.where(px == 0, p, 7 - p)
            return (px, py, my_z)

        def chunk(ref, c, d):
            c = (c + 2 * R) % R
            return ref.at[pl.ds(base + d * HALF + c * CH, CH), :]

        barrier = pltpu.get_barrier_semaphore()
        for sgn in (1, -1):
            pl.semaphore_signal(
                barrier, inc=1, device_id=coords_of(pos + sgn),
                device_id_type=pl.DeviceIdType.MESH,
            )
        pl.semaphore_signal(
            barrier, inc=1, device_id=partner,
            device_id_type=pl.DeviceIdType.MESH,
        )
        pl.semaphore_wait(barrier, 3)

        sends = []
        z_rdmas = []

        def z_forward(src, c, d, k):
            rdma = pltpu.make_async_remote_copy(
                src_ref=src,
                dst_ref=chunk(out_hbm, c, d),
                send_sem=zsend_sems.at[d, k],
                recv_sem=zrecv_sems.at[d, k],
                device_id=partner,
                device_id_type=pl.DeviceIdType.MESH,
            )
            rdma.start()
            z_rdmas.append(rdma)

        for s in range(R - 1):
            started = []
            stages = []
            for d in range(NDIR):
                sgn = 1 if d == 0 else -1
                src = (
                    chunk(x_hbm, pos, d) if s == 0 else comm.at[d, s - 1]
                )
                rdma = pltpu.make_async_remote_copy(
                    src_ref=src,
                    dst_ref=comm.at[d, s],
                    send_sem=send_sems.at[d, s],
                    recv_sem=recv_sems.at[d, s],
                    device_id=coords_of(pos + sgn),
                    device_id_type=pl.DeviceIdType.MESH,
                )
                rdma.start()
                started.append(rdma)
                sends.append(rdma)
                cp = pltpu.make_async_copy(
                    chunk(x_hbm, pos - sgn * (s + 1), d),
                    stage.at[d], local_sems.at[d],
                )
                cp.start()
                stages.append(cp)
            for d in range(NDIR):
                started[d].wait_recv()
                stages[d].wait()
                if s == 0:
                    comm[d, s] = (comm[d, s] + stage[d]) * 0.5
                else:
                    comm[d, s] = comm[d, s] + stage[d] * 0.5

        for d in range(NDIR):
            sgn = 1 if d == 0 else -1
            cp = pltpu.make_async_copy(
                comm.at[d, R - 2], chunk(out_hbm, pos + sgn, d),
                local_sems.at[d],
            )
            cp.start()
            cp.wait()
            z_forward(comm.at[d, R - 2], pos + sgn, d, 0)

        for t in range(R - 1):
            started = []
            for d in range(NDIR):
                sgn = 1 if d == 0 else -1
                c_send = pos + sgn - sgn * t
                src = comm.at[d, R - 2] if t == 0 else chunk(out_hbm, c_send, d)
                rdma = pltpu.make_async_remote_copy(
                    src_ref=src,
                    dst_ref=chunk(out_hbm, c_send, d),
                    send_sem=send_sems.at[d, R - 1 + t],
                    recv_sem=recv_sems.at[d, R - 1 + t],
                    device_id=coords_of(pos + sgn),
                    device_id_type=pl.DeviceIdType.MESH,
                )
                rdma.start()
                started.append(rdma)
                sends.append(rdma)
            for d in range(NDIR):
                sgn = 1 if d == 0 else -1
                started[d].wait_recv()
                c_recv = pos - sgn * t
                z_forward(chunk(out_hbm, c_recv, d), c_recv, d, 1 + t)

        for rdma in sends:
            rdma.wait_send()
        for rdma in z_rdmas:
            rdma.wait_send()
            rdma.wait_recv()

    n_steps = 2 * (R - 1)
    return pl.pallas_call(
        body,
        out_shape=jax.ShapeDtypeStruct((M, N), x.dtype),
        in_specs=[pl.BlockSpec(memory_space=pltpu.MemorySpace.HBM)],
        out_specs=pl.BlockSpec(memory_space=pltpu.MemorySpace.HBM),
        scratch_shapes=[
            pltpu.VMEM((NDIR, R - 1, CH, N), x.dtype),
            pltpu.VMEM((NDIR, CH, N), x.dtype),
            pltpu.SemaphoreType.DMA((NDIR, n_steps)),
            pltpu.SemaphoreType.DMA((NDIR, n_steps)),
            pltpu.SemaphoreType.DMA((NDIR, R)),
            pltpu.SemaphoreType.DMA((NDIR, R)),
            pltpu.SemaphoreType.DMA((NDIR,)),
        ],
        compiler_params=pltpu.CompilerParams(
            collective_id=0, vmem_limit_bytes=63 * 1024 * 1024
        ),
    )(x)
